# initial kernel scaffold (unmeasured)
import jax
import jax.numpy as jnp
from jax import lax
from jax.experimental import pallas as pl
from jax.experimental.pallas import tpu as pltpu

M, D = 8192, 2048


def _exchange(mine_bf16):

    def body(src_ref, dst_ref, send_sem, recv_sem):
        my_x = lax.axis_index("x")
        my_y = lax.axis_index("y")
        my_z = lax.axis_index("z")
        other = (1 - my_x, my_y, my_z)

        barrier_sem = pltpu.get_barrier_semaphore()
        pl.semaphore_signal(
            barrier_sem, inc=1, device_id=other,
            device_id_type=pl.DeviceIdType.MESH,
        )
        pl.semaphore_wait(barrier_sem, 1)

        rdma = pltpu.make_async_remote_copy(
            src_ref=src_ref,
            dst_ref=dst_ref,
            send_sem=send_sem,
            recv_sem=recv_sem,
            device_id=other,
            device_id_type=pl.DeviceIdType.MESH,
        )
        rdma.start()
        rdma.wait()

    return pl.pallas_call(
        body,
        out_shape=jax.ShapeDtypeStruct((M, D), jnp.bfloat16),
        in_specs=[pl.BlockSpec(memory_space=pltpu.ANY)],
        out_specs=pl.BlockSpec(memory_space=pltpu.ANY),
        scratch_shapes=[pltpu.SemaphoreType.DMA, pltpu.SemaphoreType.DMA],
        compiler_params=pltpu.CompilerParams(
            collective_id=0, has_side_effects=True
        ),
    )(mine_bf16)


def _compute(mine, theirs, resid, gamma2d):
    BM = 512

    def body(a_ref, b_ref, r_ref, g_ref, o_ref):
        y = (
            a_ref[...].astype(jnp.float32)
            + b_ref[...].astype(jnp.float32)
            + r_ref[...]
        )
        ms = jnp.mean(y * y, axis=-1, keepdims=True)
        o_ref[...] = y * lax.rsqrt(ms + 1e-6) * g_ref[...]

    grid = (M // BM,)
    return pl.pallas_call(
        body,
        grid=grid,
        out_shape=jax.ShapeDtypeStruct((M, D), jnp.float32),
        in_specs=[
            pl.BlockSpec((BM, D), lambda i: (i, 0)),
            pl.BlockSpec((BM, D), lambda i: (i, 0)),
            pl.BlockSpec((BM, D), lambda i: (i, 0)),
            pl.BlockSpec((1, D), lambda i: (0, 0)),
        ],
        out_specs=pl.BlockSpec((BM, D), lambda i: (i, 0)),
    )(mine, theirs, resid, gamma2d)


def kernel(partial, resid, gamma):
    mine = partial[0].astype(jnp.bfloat16)
    theirs = _exchange(mine)
    return _compute(mine, theirs, resid, gamma.reshape(1, D))


# baseline (device time: 463059 ns/iter reference)
import jax
import jax.numpy as jnp
from jax import lax
from jax.experimental import pallas as pl
from jax.experimental.pallas import tpu as pltpu

M, D = 8192, 2048


def _exchange(mine_bf16):

    def body(src_ref, dst_ref, send_sem, recv_sem):
        my_x = lax.axis_index("x")
        my_y = lax.axis_index("y")
        my_z = lax.axis_index("z")
        other = (1 - my_x, my_y, my_z)

        barrier_sem = pltpu.get_barrier_semaphore()
        pl.semaphore_signal(
            barrier_sem, inc=1, device_id=other,
            device_id_type=pl.DeviceIdType.MESH,
        )
        pl.semaphore_wait(barrier_sem, 1)

        rdma = pltpu.make_async_remote_copy(
            src_ref=src_ref,
            dst_ref=dst_ref,
            send_sem=send_sem,
            recv_sem=recv_sem,
            device_id=other,
            device_id_type=pl.DeviceIdType.MESH,
        )
        rdma.start()
        rdma.wait()

    return pl.pallas_call(
        body,
        out_shape=jax.ShapeDtypeStruct((M, D), jnp.bfloat16),
        in_specs=[pl.BlockSpec(memory_space=pl.ANY)],
        out_specs=pl.BlockSpec(memory_space=pl.ANY),
        scratch_shapes=[pltpu.SemaphoreType.DMA, pltpu.SemaphoreType.DMA],
        compiler_params=pltpu.CompilerParams(
            collective_id=0, has_side_effects=True
        ),
    )(mine_bf16)


def _compute(mine, theirs, resid, gamma2d):
    BM = 512

    def body(a_ref, b_ref, r_ref, g_ref, o_ref):
        y = (
            a_ref[...].astype(jnp.float32)
            + b_ref[...].astype(jnp.float32)
            + r_ref[...]
        )
        ms = jnp.mean(y * y, axis=-1, keepdims=True)
        o_ref[...] = y * lax.rsqrt(ms + 1e-6) * g_ref[...]

    grid = (M // BM,)
    return pl.pallas_call(
        body,
        grid=grid,
        out_shape=jax.ShapeDtypeStruct((M, D), jnp.float32),
        in_specs=[
            pl.BlockSpec((BM, D), lambda i: (i, 0)),
            pl.BlockSpec((BM, D), lambda i: (i, 0)),
            pl.BlockSpec((BM, D), lambda i: (i, 0)),
            pl.BlockSpec((1, D), lambda i: (0, 0)),
        ],
        out_specs=pl.BlockSpec((BM, D), lambda i: (i, 0)),
    )(mine, theirs, resid, gamma2d)


def kernel(partial, resid, gamma):
    mine = partial[0].astype(jnp.bfloat16)
    theirs = _exchange(mine)
    return _compute(mine, theirs, resid, gamma.reshape(1, D))


# device time: 272888 ns/iter; 1.6969x vs baseline; 1.6969x over previous
import jax
import jax.numpy as jnp
from jax import lax
from jax.experimental import pallas as pl
from jax.experimental.pallas import tpu as pltpu

M, D = 8192, 2048
NP = 16
CH = M // NP
NR = 8
NL = 7

RING = [(0, 0), (0, 1), (0, 2), (0, 3),
        (1, 3), (1, 2), (1, 1),
        (2, 1), (2, 2), (2, 3),
        (3, 3), (3, 2), (3, 1), (3, 0),
        (2, 0), (1, 0)]

POS_T = [0] * 16
RNY_T = [0] * 16
RNZ_T = [0] * 16
LNY_T = [0] * 16
LNZ_T = [0] * 16
for _p, (_y, _z) in enumerate(RING):
    _i = _y * 4 + _z
    POS_T[_i] = _p
    RNY_T[_i], RNZ_T[_i] = RING[(_p + 1) % NP]
    LNY_T[_i], LNZ_T[_i] = RING[(_p - 1) % NP]


def kernel(partial, resid, gamma):
    def body(p_ref, r_ref, g_ref, o_ref,
             comm, myc, mybf, prt, rba, rbb, oba, obb,
             sem_my, sem_xs, sem_xr, s_r, r_r, s_l, r_l,
             sem_ra, sem_rb, sem_oa, sem_ob):
        my_x = lax.axis_index("x")
        my_y = lax.axis_index("y")
        my_z = lax.axis_index("z")
        idx = my_y * 4 + my_z

        def sel(tbl):
            acc = jnp.full((), tbl[0], jnp.int32)
            for i in range(1, 16):
                acc = jnp.where(idx == i, jnp.int32(tbl[i]), acc)
            return acc

        k = sel(POS_T)
        right = (my_x, sel(RNY_T), sel(RNZ_T))
        left = (my_x, sel(LNY_T), sel(LNZ_T))
        partner = (1 - my_x, my_y, my_z)

        cp_my = pltpu.make_async_copy(
            p_ref.at[0, pl.ds(k * CH, CH)], myc, sem_my)
        cp_my.start()
        cp_ra = pltpu.make_async_copy(
            r_ref.at[pl.ds(k * CH, CH)], rba, sem_ra)
        cp_ra.start()

        bar = pltpu.get_barrier_semaphore()
        for nbr in (partner, left, right):
            pl.semaphore_signal(bar, inc=1, device_id=nbr,
                                device_id_type=pl.DeviceIdType.MESH)
        pl.semaphore_wait(bar, 3)

        cp_my.wait()
        mybf[...] = myc[...].astype(jnp.bfloat16)

        rdma_x = pltpu.make_async_remote_copy(
            src_ref=mybf, dst_ref=prt, send_sem=sem_xs, recv_sem=sem_xr,
            device_id=partner, device_id_type=pl.DeviceIdType.MESH)
        rdma_x.start()
        rdma_x.wait()

        own_s = myc[...] + prt[...].astype(jnp.float32)
        comm[pl.ds(k * CH, CH), :] = own_s.astype(jnp.bfloat16)

        def rsend(h):
            c = (k - h) % NP
            return pltpu.make_async_remote_copy(
                src_ref=comm.at[pl.ds(c * CH, CH)],
                dst_ref=comm.at[pl.ds(c * CH, CH)],
                send_sem=s_r.at[h], recv_sem=r_r.at[h],
                device_id=right, device_id_type=pl.DeviceIdType.MESH)

        def rrecv(h):
            c = (k - 1 - h) % NP
            return pltpu.make_async_remote_copy(
                src_ref=comm.at[pl.ds(c * CH, CH)],
                dst_ref=comm.at[pl.ds(c * CH, CH)],
                send_sem=s_r.at[h], recv_sem=r_r.at[h],
                device_id=left, device_id_type=pl.DeviceIdType.MESH)

        def lsend(h):
            c = (k + h) % NP
            return pltpu.make_async_remote_copy(
                src_ref=comm.at[pl.ds(c * CH, CH)],
                dst_ref=comm.at[pl.ds(c * CH, CH)],
                send_sem=s_l.at[h], recv_sem=r_l.at[h],
                device_id=left, device_id_type=pl.DeviceIdType.MESH)

        def lrecv(h):
            c = (k + 1 + h) % NP
            return pltpu.make_async_remote_copy(
                src_ref=comm.at[pl.ds(c * CH, CH)],
                dst_ref=comm.at[pl.ds(c * CH, CH)],
                send_sem=s_l.at[h], recv_sem=r_l.at[h],
                device_id=right, device_id_type=pl.DeviceIdType.MESH)

        pending = {"a": None, "b": None}

        def process(c, y_src, rbuf, obuf, osem, key):
            if pending[key] is not None:
                pending[key].wait()
            y = y_src + rbuf[...]
            ms = jnp.mean(y * y, axis=-1, keepdims=True)
            obuf[...] = y * lax.rsqrt(ms + 1e-6) * g_ref[...]
            cp = pltpu.make_async_copy(
                obuf, o_ref.at[pl.ds(c * CH, CH)], osem)
            cp.start()
            pending[key] = cp

        sends = []
        s0 = rsend(0)
        s0.start()
        sends.append(s0)
        s0 = lsend(0)
        s0.start()
        sends.append(s0)

        cp_ra.wait()
        process(k, own_s, rba, oba, sem_oa, "a")
        cp_ra = pltpu.make_async_copy(
            r_ref.at[pl.ds(((k - 1) % NP) * CH, CH)], rba, sem_ra)
        cp_ra.start()
        cp_rb = pltpu.make_async_copy(
            r_ref.at[pl.ds(((k + 1) % NP) * CH, CH)], rbb, sem_rb)
        cp_rb.start()

        for h in range(NR):
            rrecv(h).wait_recv()
            if h + 1 < NR:
                s = rsend(h + 1)
                s.start()
                sends.append(s)
            if h < NL:
                lrecv(h).wait_recv()
                if h + 1 < NL:
                    s = lsend(h + 1)
                    s.start()
                    sends.append(s)

            cR = (k - 1 - h) % NP
            cp_ra.wait()
            process(cR, comm[pl.ds(cR * CH, CH), :].astype(jnp.float32),
                    rba, oba, sem_oa, "a")
            if h + 1 < NR:
                cp_ra = pltpu.make_async_copy(
                    r_ref.at[pl.ds(((k - 2 - h) % NP) * CH, CH)],
                    rba, sem_ra)
                cp_ra.start()
            if h < NL:
                cL = (k + 1 + h) % NP
                cp_rb.wait()
                process(cL, comm[pl.ds(cL * CH, CH), :].astype(jnp.float32),
                        rbb, obb, sem_ob, "b")
                if h + 1 < NL:
                    cp_rb = pltpu.make_async_copy(
                        r_ref.at[pl.ds(((k + 2 + h) % NP) * CH, CH)],
                        rbb, sem_rb)
                    cp_rb.start()

        for s in sends:
            s.wait_send()
        pending["a"].wait()
        pending["b"].wait()

    return pl.pallas_call(
        body,
        out_shape=jax.ShapeDtypeStruct((M, D), jnp.float32),
        in_specs=[
            pl.BlockSpec(memory_space=pl.ANY),
            pl.BlockSpec(memory_space=pl.ANY),
            pl.BlockSpec(memory_space=pltpu.MemorySpace.VMEM),
        ],
        out_specs=pl.BlockSpec(memory_space=pl.ANY),
        scratch_shapes=[
            pltpu.VMEM((M, D), jnp.bfloat16),
            pltpu.VMEM((CH, D), jnp.float32),
            pltpu.VMEM((CH, D), jnp.bfloat16),
            pltpu.VMEM((CH, D), jnp.bfloat16),
            pltpu.VMEM((CH, D), jnp.float32),
            pltpu.VMEM((CH, D), jnp.float32),
            pltpu.VMEM((CH, D), jnp.float32),
            pltpu.VMEM((CH, D), jnp.float32),
            pltpu.SemaphoreType.DMA,
            pltpu.SemaphoreType.DMA,
            pltpu.SemaphoreType.DMA,
            pltpu.SemaphoreType.DMA((NR,)),
            pltpu.SemaphoreType.DMA((NR,)),
            pltpu.SemaphoreType.DMA((NL,)),
            pltpu.SemaphoreType.DMA((NL,)),
            pltpu.SemaphoreType.DMA,
            pltpu.SemaphoreType.DMA,
            pltpu.SemaphoreType.DMA,
            pltpu.SemaphoreType.DMA,
        ],
        compiler_params=pltpu.CompilerParams(
            collective_id=0, has_side_effects=True,
            vmem_limit_bytes=100 * 1024 * 1024),
    )(partial, resid, gamma.reshape(1, D))
